# baseline (speedup 1.0000x reference)
import functools

import jax
import jax.numpy as jnp
from jax import lax
from jax.experimental import pallas as pl
from jax.experimental.pallas import tpu as pltpu
from jax.experimental.pallas import tpu_sc as plsc

N_NODES = 10000
N_EDGES = 320000
IN_DIM = 128
L1_DIM = 16
N_TYPES = 1000

N_CORES = 2
N_SUBCORES = 16
N_WORKERS = N_CORES * N_SUBCORES

EPW = N_EDGES // N_WORKERS
CHUNK = 80
NBUF = 5
N_CHUNKS = EPW // CHUNK
GR = CHUNK // 16

_SC_MESH = plsc.VectorSubcoreMesh(core_axis_name="c", subcore_axis_name="s")
_SC_PARAMS = pltpu.CompilerParams(
    needs_layout_passes=False, use_tc_tiling_on_sc=False)


def _proj_body(z_ref, w1_ref, w2_ref, p1_ref, p2_ref):
    zz = z_ref[...]
    p1_ref[...] = jnp.maximum(
        jnp.dot(zz, w1_ref[...], preferred_element_type=jnp.float32), 0.0)
    p2_ref[...] = jnp.maximum(
        jnp.dot(zz, w2_ref[...], preferred_element_type=jnp.float32), 0.0)


def _project(z, w1_l1, w2_l1):
    rows = 1000
    grid = (N_NODES // rows,)
    return pl.pallas_call(
        _proj_body,
        grid=grid,
        in_specs=[
            pl.BlockSpec((rows, IN_DIM), lambda i: (i, 0)),
            pl.BlockSpec((IN_DIM, L1_DIM), lambda i: (0, 0)),
            pl.BlockSpec((IN_DIM, L1_DIM), lambda i: (0, 0)),
        ],
        out_specs=[
            pl.BlockSpec((rows, L1_DIM), lambda i: (i, 0)),
            pl.BlockSpec((rows, L1_DIM), lambda i: (i, 0)),
        ],
        out_shape=[
            jax.ShapeDtypeStruct((N_NODES, L1_DIM), jnp.float32),
            jax.ShapeDtypeStruct((N_NODES, L1_DIM), jnp.float32),
        ],
    )(z, w1_l1, w2_l1)


def _edge_body(p1_hbm, p2_hbm, v1_hbm, v2_hbm, src_hbm, dst_hbm, et_hbm,
               out_hbm,
               v1_v, v2_v, s_v, d_v, t_v,
               p1r0, p1r1, p1r2, p1r3, p1r4,
               p2r0, p2r1, p2r2, p2r3, p2r4,
               outb_v, sem, osem):
    wid = lax.axis_index("s") * N_CORES + lax.axis_index("c")
    base = wid * EPW
    P1R = [p1r0, p1r1, p1r2, p1r3, p1r4]
    P2R = [p2r0, p2r1, p2r2, p2r3, p2r4]

    pltpu.sync_copy(v1_hbm, v1_v)
    pltpu.sync_copy(v2_hbm, v2_v)
    pltpu.sync_copy(src_hbm.at[pl.ds(base, EPW)], s_v)
    pltpu.sync_copy(dst_hbm.at[pl.ds(base, EPW)], d_v)
    pltpu.sync_copy(et_hbm.at[pl.ds(base, EPW)], t_v)

    def fire(coff, b):
        sl = pl.ds(coff, CHUNK)
        return [pltpu.async_copy(p1_hbm.at[s_v.at[sl]], P1R[b], sem),
                pltpu.async_copy(p2_hbm.at[d_v.at[sl]], P2R[b], sem)]

    for b in range(NBUF):
        fire(b * CHUNK, b)

    @pl.loop(0, N_CHUNKS, step=NBUF)
    def chunk_span(c0):
        for b in range(NBUF):
            coff = (c0 + b) * CHUNK
            sl = pl.ds(coff, CHUNK)
            pltpu.make_async_copy(p1_hbm.at[s_v.at[sl]], P1R[b], sem).wait()
            pltpu.make_async_copy(p2_hbm.at[d_v.at[sl]], P2R[b], sem).wait()

            @plsc.parallel_loop(0, GR, unroll=1)
            def group_body(g):
                sl16 = pl.ds(coff + g * 16, 16)
                rows = lax.iota(jnp.int32, 16) + g * 16
                tid = t_v[sl16]
                acc1 = jnp.zeros((16,), jnp.float32)
                acc2 = jnp.zeros((16,), jnp.float32)
                for j in range(L1_DIM):
                    jcol = jnp.full((16,), j, jnp.int32)
                    a = plsc.load_gather(P1R[b], [rows, jcol])
                    bb = plsc.load_gather(v1_v, [tid, jcol])
                    cc = plsc.load_gather(P2R[b], [rows, jcol])
                    dd = plsc.load_gather(v2_v, [tid, jcol])
                    acc1 = acc1 + a * bb
                    acc2 = acc2 + cc * dd
                acc = acc1 + acc2
                outb_v[sl16] = 1.0 / (1.0 + jnp.exp(-acc))

            @pl.when(c0 + b + NBUF < N_CHUNKS)
            def _():
                fire((c0 + b + NBUF) * CHUNK, b)

            pltpu.async_copy(outb_v.at[pl.ds(coff, CHUNK)],
                             out_hbm.at[pl.ds(base + coff, CHUNK)], osem)

    pltpu.make_async_copy(outb_v, out_hbm.at[pl.ds(base, EPW)], osem).wait()


_edge_kernel = functools.partial(
    pl.kernel,
    out_type=jax.ShapeDtypeStruct((N_EDGES,), jnp.float32),
    mesh=_SC_MESH,
    compiler_params=_SC_PARAMS,
    scratch_types=(
        [pltpu.VMEM((N_TYPES, L1_DIM), jnp.float32) for _ in range(2)]
        + [pltpu.VMEM((EPW,), jnp.int32) for _ in range(3)]
        + [pltpu.VMEM((CHUNK, L1_DIM), jnp.float32) for _ in range(2 * NBUF)]
        + [pltpu.VMEM((EPW,), jnp.float32)]
        + [pltpu.SemaphoreType.DMA, pltpu.SemaphoreType.DMA]
    ),
)(_edge_body)


def kernel(z, edge_index, edge_type, w1_l1, w1_l2, w2_l1, w2_l2):
    p1, p2 = _project(z, w1_l1, w2_l1)
    src = edge_index[0].astype(jnp.int32)
    dst = edge_index[1].astype(jnp.int32)
    et = edge_type.astype(jnp.int32)
    return _edge_kernel(p1, p2, w1_l2, w2_l2, src, dst, et)


# speedup vs baseline: 1.9873x; 1.9873x over previous
import functools

import jax
import jax.numpy as jnp
from jax import lax
from jax.experimental import pallas as pl
from jax.experimental.pallas import tpu as pltpu
from jax.experimental.pallas import tpu_sc as plsc

N_NODES = 10000
N_EDGES = 320000
IN_DIM = 128
L1_DIM = 16
N_TYPES = 1000
T_PAD = 1024

N_CORES = 2
N_SUBCORES = 16
N_WORKERS = N_CORES * N_SUBCORES

EPW = N_EDGES // N_WORKERS
CHUNK = 2000
N_CHUNKS = EPW // CHUNK
GROUPS_W = EPW // 16
GROUPS_C = CHUNK // 16

_CONTRACT_MINOR = (((1,), (1,)), ((), ()))

_SC_MESH = plsc.VectorSubcoreMesh(core_axis_name="c", subcore_axis_name="s")
_SC_PARAMS = pltpu.CompilerParams(
    needs_layout_passes=False, use_tc_tiling_on_sc=False)


def _score_body(z_ref, w_ref, v_ref, q_ref):
    p = jnp.maximum(
        jnp.dot(z_ref[...], w_ref[...], preferred_element_type=jnp.float32),
        0.0)
    q = lax.dot_general(p, v_ref[...], _CONTRACT_MINOR,
                        preferred_element_type=jnp.float32)
    for j in range(T_PAD // 128):
        q_ref[:, j, :] = q[:, 128 * j:128 * (j + 1)]


def _scores(z, w_l1, w_l2):
    rows = 1000
    grid = (N_NODES // rows,)
    vp = jnp.pad(w_l2, ((0, T_PAD - N_TYPES), (0, 0)))
    q = pl.pallas_call(
        _score_body,
        grid=grid,
        in_specs=[
            pl.BlockSpec((rows, IN_DIM), lambda i: (i, 0)),
            pl.BlockSpec((IN_DIM, L1_DIM), lambda i: (0, 0)),
            pl.BlockSpec((T_PAD, L1_DIM), lambda i: (0, 0)),
        ],
        out_specs=pl.BlockSpec((rows, T_PAD // 128, 128), lambda i: (i, 0, 0)),
        out_shape=jax.ShapeDtypeStruct((N_NODES, T_PAD // 128, 128),
                                       jnp.float32),
    )(z, w_l1, vp)
    return q.reshape(N_NODES * T_PAD)


def _idx_body(src_hbm, dst_hbm, et_hbm, i1_hbm, i2_hbm,
              s_v, d_v, t_v, i1_v, i2_v):
    wid = lax.axis_index("s") * N_CORES + lax.axis_index("c")
    base = wid * EPW
    pltpu.sync_copy(src_hbm.at[pl.ds(base, EPW)], s_v)
    pltpu.sync_copy(dst_hbm.at[pl.ds(base, EPW)], d_v)
    pltpu.sync_copy(et_hbm.at[pl.ds(base, EPW)], t_v)

    @plsc.parallel_loop(0, GROUPS_W, unroll=4)
    def idx_body(g):
        sl = pl.ds(g * 16, 16)
        t16 = t_v[sl]
        i1_v[sl] = s_v[sl] * T_PAD + t16
        i2_v[sl] = d_v[sl] * T_PAD + t16

    pltpu.sync_copy(i1_v, i1_hbm.at[pl.ds(base, EPW)])
    pltpu.sync_copy(i2_v, i2_hbm.at[pl.ds(base, EPW)])


_idx_kernel = functools.partial(
    pl.kernel,
    out_type=[jax.ShapeDtypeStruct((N_EDGES,), jnp.int32),
              jax.ShapeDtypeStruct((N_EDGES,), jnp.int32)],
    mesh=_SC_MESH,
    compiler_params=_SC_PARAMS,
    scratch_types=[pltpu.VMEM((EPW,), jnp.int32) for _ in range(5)],
)(_idx_body)


def _gather1_body(q1_hbm, i1_hbm, part_hbm, i1_v, q1r_v, sem, osem):
    wid = lax.axis_index("s") * N_CORES + lax.axis_index("c")
    base = wid * EPW
    pltpu.sync_copy(i1_hbm.at[pl.ds(base, EPW)], i1_v)
    pend = []
    for c in range(N_CHUNKS):
        sl = pl.ds(c * CHUNK, CHUNK)
        pend.append(pltpu.async_copy(q1_hbm.at[i1_v.at[sl]], q1r_v.at[sl],
                                     sem))
    outs = []
    for c in range(N_CHUNKS):
        pend[c].wait()
        outs.append(pltpu.async_copy(
            q1r_v.at[pl.ds(c * CHUNK, CHUNK)],
            part_hbm.at[pl.ds(base + c * CHUNK, CHUNK)], osem))
    for cp in outs:
        cp.wait()


_gather1_kernel = functools.partial(
    pl.kernel,
    out_type=jax.ShapeDtypeStruct((N_EDGES,), jnp.float32),
    mesh=_SC_MESH,
    compiler_params=_SC_PARAMS,
    scratch_types=[
        pltpu.VMEM((EPW,), jnp.int32),
        pltpu.VMEM((EPW,), jnp.float32),
        pltpu.SemaphoreType.DMA,
        pltpu.SemaphoreType.DMA,
    ],
)(_gather1_body)


def _gather2_body(q2_hbm, i2_hbm, part_hbm, out_hbm,
                  i2_v, q2r_v, partb_v, outb_v, sem, osem):
    wid = lax.axis_index("s") * N_CORES + lax.axis_index("c")
    base = wid * EPW
    pltpu.sync_copy(i2_hbm.at[pl.ds(base, EPW)], i2_v)
    pend = []
    for c in range(N_CHUNKS):
        sl = pl.ds(c * CHUNK, CHUNK)
        pend.append(pltpu.async_copy(q2_hbm.at[i2_v.at[sl]], q2r_v.at[sl],
                                     sem))
    pltpu.sync_copy(part_hbm.at[pl.ds(base, EPW)], partb_v)
    outs = []
    for c in range(N_CHUNKS):
        pend[c].wait()

        @plsc.parallel_loop(0, GROUPS_C, unroll=2)
        def out_body(g):
            sl = pl.ds(c * CHUNK + g * 16, 16)
            acc = partb_v[sl] + q2r_v[sl]
            outb_v[sl] = 1.0 / (1.0 + jnp.exp(-acc))

        outs.append(pltpu.async_copy(
            outb_v.at[pl.ds(c * CHUNK, CHUNK)],
            out_hbm.at[pl.ds(base + c * CHUNK, CHUNK)], osem))
    for cp in outs:
        cp.wait()


_gather2_kernel = functools.partial(
    pl.kernel,
    out_type=jax.ShapeDtypeStruct((N_EDGES,), jnp.float32),
    mesh=_SC_MESH,
    compiler_params=_SC_PARAMS,
    scratch_types=[
        pltpu.VMEM((EPW,), jnp.int32),
        pltpu.VMEM((EPW,), jnp.float32),
        pltpu.VMEM((EPW,), jnp.float32),
        pltpu.VMEM((EPW,), jnp.float32),
        pltpu.SemaphoreType.DMA,
        pltpu.SemaphoreType.DMA,
    ],
)(_gather2_body)


def kernel(z, edge_index, edge_type, w1_l1, w1_l2, w2_l1, w2_l2):
    src = edge_index[0].astype(jnp.int32)
    dst = edge_index[1].astype(jnp.int32)
    et = edge_type.astype(jnp.int32)
    i1, i2 = _idx_kernel(src, dst, et)
    q1 = _scores(z, w1_l1, w1_l2)
    part = _gather1_kernel(q1, i1)
    q2 = _scores(z, w2_l1, w2_l2)
    return _gather2_kernel(q2, i2, part)


# speedup vs baseline: 2.0020x; 1.0074x over previous
import functools

import jax
import jax.numpy as jnp
from jax import lax
from jax.experimental import pallas as pl
from jax.experimental.pallas import tpu as pltpu
from jax.experimental.pallas import tpu_sc as plsc

N_NODES = 10000
N_EDGES = 320000
IN_DIM = 128
L1_DIM = 16
N_TYPES = 1000
T_PAD = 1024

N_CORES = 2
N_SUBCORES = 16
N_WORKERS = N_CORES * N_SUBCORES

EPW = N_EDGES // N_WORKERS
CHUNK = 2000
N_CHUNKS = EPW // CHUNK
GROUPS_W = EPW // 16
GROUPS_C = CHUNK // 16

_CONTRACT_MINOR = (((1,), (1,)), ((), ()))

_SC_MESH = plsc.VectorSubcoreMesh(core_axis_name="c", subcore_axis_name="s")
_SC_PARAMS = pltpu.CompilerParams(
    needs_layout_passes=False, use_tc_tiling_on_sc=False)


def _score_body(z_ref, w_ref, v_ref, q_ref):
    p = jnp.maximum(
        jnp.dot(z_ref[...], w_ref[...], preferred_element_type=jnp.float32),
        0.0)
    q = lax.dot_general(p, v_ref[...], _CONTRACT_MINOR,
                        preferred_element_type=jnp.float32)
    for j in range(T_PAD // 128):
        q_ref[:, j, :] = q[:, 128 * j:128 * (j + 1)]


def _scores(z, w_l1, w_l2):
    rows = 2000
    grid = (N_NODES // rows,)
    vp = jnp.pad(w_l2, ((0, T_PAD - N_TYPES), (0, 0)))
    q = pl.pallas_call(
        _score_body,
        grid=grid,
        in_specs=[
            pl.BlockSpec((rows, IN_DIM), lambda i: (i, 0)),
            pl.BlockSpec((IN_DIM, L1_DIM), lambda i: (0, 0)),
            pl.BlockSpec((T_PAD, L1_DIM), lambda i: (0, 0)),
        ],
        out_specs=pl.BlockSpec((rows, T_PAD // 128, 128), lambda i: (i, 0, 0)),
        out_shape=jax.ShapeDtypeStruct((N_NODES, T_PAD // 128, 128),
                                       jnp.float32),
    )(z, w_l1, vp)
    return q.reshape(N_NODES * T_PAD)


def _idx_body(src_hbm, dst_hbm, et_hbm, i1_hbm, i2_hbm,
              s_v, d_v, t_v, i1_v, i2_v):
    wid = lax.axis_index("s") * N_CORES + lax.axis_index("c")
    base = wid * EPW
    pltpu.sync_copy(src_hbm.at[pl.ds(base, EPW)], s_v)
    pltpu.sync_copy(dst_hbm.at[pl.ds(base, EPW)], d_v)
    pltpu.sync_copy(et_hbm.at[pl.ds(base, EPW)], t_v)

    @plsc.parallel_loop(0, GROUPS_W, unroll=4)
    def idx_body(g):
        sl = pl.ds(g * 16, 16)
        t16 = t_v[sl]
        i1_v[sl] = s_v[sl] * T_PAD + t16
        i2_v[sl] = d_v[sl] * T_PAD + t16

    pltpu.sync_copy(i1_v, i1_hbm.at[pl.ds(base, EPW)])
    pltpu.sync_copy(i2_v, i2_hbm.at[pl.ds(base, EPW)])


_idx_kernel = functools.partial(
    pl.kernel,
    out_type=[jax.ShapeDtypeStruct((N_EDGES,), jnp.int32),
              jax.ShapeDtypeStruct((N_EDGES,), jnp.int32)],
    mesh=_SC_MESH,
    compiler_params=_SC_PARAMS,
    scratch_types=[pltpu.VMEM((EPW,), jnp.int32) for _ in range(5)],
)(_idx_body)


def _gather1_body(q1_hbm, i1_hbm, part_hbm, i1_v, q1r_v, sem, osem):
    wid = lax.axis_index("s") * N_CORES + lax.axis_index("c")
    base = wid * EPW
    pltpu.sync_copy(i1_hbm.at[pl.ds(base, EPW)], i1_v)
    pend = []
    for c in range(N_CHUNKS):
        sl = pl.ds(c * CHUNK, CHUNK)
        pend.append(pltpu.async_copy(q1_hbm.at[i1_v.at[sl]], q1r_v.at[sl],
                                     sem))
    outs = []
    for c in range(N_CHUNKS):
        pend[c].wait()
        outs.append(pltpu.async_copy(
            q1r_v.at[pl.ds(c * CHUNK, CHUNK)],
            part_hbm.at[pl.ds(base + c * CHUNK, CHUNK)], osem))
    for cp in outs:
        cp.wait()


_gather1_kernel = functools.partial(
    pl.kernel,
    out_type=jax.ShapeDtypeStruct((N_EDGES,), jnp.float32),
    mesh=_SC_MESH,
    compiler_params=_SC_PARAMS,
    scratch_types=[
        pltpu.VMEM((EPW,), jnp.int32),
        pltpu.VMEM((EPW,), jnp.float32),
        pltpu.SemaphoreType.DMA,
        pltpu.SemaphoreType.DMA,
    ],
)(_gather1_body)


def _gather2_body(q2_hbm, i2_hbm, part_hbm, out_hbm,
                  i2_v, q2r_v, partb_v, outb_v, sem, osem):
    wid = lax.axis_index("s") * N_CORES + lax.axis_index("c")
    base = wid * EPW
    pltpu.sync_copy(i2_hbm.at[pl.ds(base, EPW)], i2_v)
    pend = []
    for c in range(N_CHUNKS):
        sl = pl.ds(c * CHUNK, CHUNK)
        pend.append(pltpu.async_copy(q2_hbm.at[i2_v.at[sl]], q2r_v.at[sl],
                                     sem))
    pltpu.sync_copy(part_hbm.at[pl.ds(base, EPW)], partb_v)
    outs = []
    for c in range(N_CHUNKS):
        pend[c].wait()

        @plsc.parallel_loop(0, GROUPS_C, unroll=2)
        def out_body(g):
            sl = pl.ds(c * CHUNK + g * 16, 16)
            acc = partb_v[sl] + q2r_v[sl]
            outb_v[sl] = 1.0 / (1.0 + jnp.exp(-acc))

        outs.append(pltpu.async_copy(
            outb_v.at[pl.ds(c * CHUNK, CHUNK)],
            out_hbm.at[pl.ds(base + c * CHUNK, CHUNK)], osem))
    for cp in outs:
        cp.wait()


_gather2_kernel = functools.partial(
    pl.kernel,
    out_type=jax.ShapeDtypeStruct((N_EDGES,), jnp.float32),
    mesh=_SC_MESH,
    compiler_params=_SC_PARAMS,
    scratch_types=[
        pltpu.VMEM((EPW,), jnp.int32),
        pltpu.VMEM((EPW,), jnp.float32),
        pltpu.VMEM((EPW,), jnp.float32),
        pltpu.VMEM((EPW,), jnp.float32),
        pltpu.SemaphoreType.DMA,
        pltpu.SemaphoreType.DMA,
    ],
)(_gather2_body)


def kernel(z, edge_index, edge_type, w1_l1, w1_l2, w2_l1, w2_l2):
    src = edge_index[0].astype(jnp.int32)
    dst = edge_index[1].astype(jnp.int32)
    et = edge_type.astype(jnp.int32)
    i1, i2 = _idx_kernel(src, dst, et)
    q1 = _scores(z, w1_l1, w1_l2)
    part = _gather1_kernel(q1, i1)
    q2 = _scores(z, w2_l1, w2_l2)
    return _gather2_kernel(q2, i2, part)
